# Initial kernel scaffold; baseline (speedup 1.0000x reference)
import functools

import jax
import jax.numpy as jnp
from jax import lax
from jax.experimental import pallas as pl
from jax.experimental.pallas import tpu as pltpu
from jax.experimental.pallas import tpu_sc as plsc

N = 10000
E = 320000
D = 128
H = 8
C = 16
HC = H * C
TW = HC + 16

NB = 400
NBLK = N // NB

ECH = 128
NCHUNK = E // ECH
NWORK = 32
_BASE_CH = NCHUNK // NWORK
_REM_CH = NCHUNK - _BASE_CH * NWORK
RPS = N // 16


def _prep_body(x_ref, w_ref, as_ref, ad_ref, tsrc_ref, tdst_ref, pmax_ref):
    h = jnp.dot(x_ref[...], w_ref[...], precision=lax.Precision.HIGHEST)
    h3 = h.reshape(NB, H, C)
    a_s = jnp.sum(h3 * as_ref[...][None], axis=-1)
    a_d = jnp.sum(h3 * ad_ref[...][None], axis=-1)
    hc = h3.transpose(0, 2, 1).reshape(NB, HC)
    zeros8 = jnp.zeros((NB, 8), jnp.float32)
    tsrc_ref[...] = jnp.concatenate([hc, a_s, zeros8], axis=1)
    tdst_ref[...] = jnp.concatenate([a_d, zeros8], axis=1)
    pmax_ref[...] = jnp.concatenate(
        [jnp.max(a_s, axis=0), jnp.max(a_d, axis=0)]).reshape(1, 1, 16)


def _prep(x, W, att_s, att_d):
    return pl.pallas_call(
        _prep_body,
        grid=(NBLK,),
        in_specs=[
            pl.BlockSpec((NB, D), lambda i: (i, 0)),
            pl.BlockSpec((D, HC), lambda i: (0, 0)),
            pl.BlockSpec((H, C), lambda i: (0, 0)),
            pl.BlockSpec((H, C), lambda i: (0, 0)),
        ],
        out_specs=[
            pl.BlockSpec((NB, TW), lambda i: (i, 0)),
            pl.BlockSpec((NB, 16), lambda i: (i, 0)),
            pl.BlockSpec((1, 1, 16), lambda i: (i, 0, 0)),
        ],
        out_shape=[
            jax.ShapeDtypeStruct((N, TW), jnp.float32),
            jax.ShapeDtypeStruct((N, 16), jnp.float32),
            jax.ShapeDtypeStruct((NBLK, 1, 16), jnp.float32),
        ],
    )(x, W, att_s, att_d)


def _sc_body(tsrc_hbm, tdst_hbm, src_hbm, dst_hbm, m_hbm, out_hbm,
             idx_s, idx_d, rows, drows, mvec, acc):
    cid = lax.axis_index("c")
    sid = lax.axis_index("s")
    wid = sid * 2 + cid

    zero16 = jnp.zeros((16,), jnp.float32)

    @pl.loop(0, ECH)
    def _(r):
        for k in range(TW // 16):
            rows[r, pl.ds(16 * k, 16)] = zero16

    @pl.loop(0, 5)
    def _(z):
        pltpu.sync_copy(rows.at[pl.ds(0, RPS // 5)],
                        acc.at[pl.ds(sid * RPS + z * (RPS // 5), RPS // 5)])

    plsc.subcore_barrier()

    pltpu.sync_copy(m_hbm, mvec)
    m = mvec[...]
    pat = lax.rem(lax.iota(jnp.int32, 16), jnp.full((16,), 8, jnp.int32))
    colv = pat + jnp.full((16,), HC, jnp.int32)

    nch = _BASE_CH + jnp.where(wid < _REM_CH, 1, 0)

    @pl.loop(0, nch)
    def _(j):
        off = (wid + NWORK * j) * ECH
        pltpu.sync_copy(src_hbm.at[pl.ds(off, ECH)], idx_s)
        pltpu.sync_copy(dst_hbm.at[pl.ds(off, ECH)], idx_d)
        pltpu.sync_copy(tsrc_hbm.at[idx_s], rows)
        pltpu.sync_copy(tdst_hbm.at[idx_d], drows)

        @pl.loop(0, ECH)
        def _(e):
            a_s = rows[e, pl.ds(HC, 16)]
            a_d = drows[e, pl.ds(0, 16)]
            t = a_s + a_d
            lrelu = jnp.maximum(t, 0.2 * t)
            wv = jnp.exp(lrelu - m)
            rows[e, pl.ds(HC, 16)] = wv
            rowv = jnp.full((16,), e, jnp.int32)
            wt = plsc.load_gather(rows, [rowv, colv])
            for k in range(H):
                sl = pl.ds(16 * k, 16)
                rows[e, sl] = rows[e, sl] * wt

        pltpu.sync_copy(rows, acc.at[idx_d], add=True)

    plsc.subcore_barrier()

    @pl.loop(0, 5)
    def _(z):
        r0 = sid * RPS + z * (RPS // 5)
        pltpu.sync_copy(acc.at[pl.ds(r0, RPS // 5)],
                        out_hbm.at[cid, pl.ds(r0, RPS // 5)])


def _sc_edges(tsrc, tdst, src, dst, m16):
    return pl.kernel(
        _sc_body,
        out_type=jax.ShapeDtypeStruct((2, N, TW), jnp.float32),
        mesh=plsc.VectorSubcoreMesh(core_axis_name="c", subcore_axis_name="s"),
        scratch_types=[
            pltpu.VMEM((ECH,), jnp.int32),
            pltpu.VMEM((ECH,), jnp.int32),
            pltpu.VMEM((ECH, TW), jnp.float32),
            pltpu.VMEM((ECH, 16), jnp.float32),
            pltpu.VMEM((16,), jnp.float32),
            pltpu.VMEM_SHARED((N, TW), jnp.float32),
        ],
    )(tsrc, tdst, src, dst, m16)


def _final_body(p0_ref, p1_ref, tsrc_ref, tdst_ref, m_ref, b_ref, o_ref):
    tsrc = tsrc_ref[...]
    asrc = tsrc[:, HC:HC + 8]
    adst = tdst_ref[...][:, :8]
    t = asrc + adst
    lrelu = jnp.maximum(t, 0.2 * t)
    wself = jnp.exp(lrelu - m_ref[0, :8][None, :])
    p0 = p0_ref[...]
    p1 = p1_ref[...]
    num = p0[:, :HC] + p1[:, :HC] + tsrc[:, :HC] * jnp.tile(wself, (1, C))
    den = p0[:, HC:HC + 8] + p1[:, HC:HC + 8] + wself + 1e-16
    outc = num / jnp.tile(den, (1, C))
    o_ref[...] = (outc.reshape(NB, C, H).transpose(0, 2, 1).reshape(NB, HC)
                  + b_ref[0][None, :])


def _final(p0, p1, tsrc, tdst, m16, bias):
    return pl.pallas_call(
        _final_body,
        grid=(NBLK,),
        in_specs=[
            pl.BlockSpec((NB, TW), lambda i: (i, 0)),
            pl.BlockSpec((NB, TW), lambda i: (i, 0)),
            pl.BlockSpec((NB, TW), lambda i: (i, 0)),
            pl.BlockSpec((NB, 16), lambda i: (i, 0)),
            pl.BlockSpec((1, 16), lambda i: (0, 0)),
            pl.BlockSpec((1, HC), lambda i: (0, 0)),
        ],
        out_specs=pl.BlockSpec((NB, HC), lambda i: (i, 0)),
        out_shape=jax.ShapeDtypeStruct((N, HC), jnp.float32),
    )(p0, p1, tsrc, tdst, m16, bias)


def kernel(x, edge_index, W, att_src, att_dst, bias):
    tsrc, tdst, pmax = _prep(x, W, att_src.reshape(H, C), att_dst.reshape(H, C))
    pm = pmax.reshape(NBLK, 16)
    m8 = jnp.max(pm[:, :8], axis=0) + jnp.max(pm[:, 8:], axis=0)
    m16 = jnp.concatenate([m8, jnp.full((8,), 1e30, jnp.float32)])
    partials = _sc_edges(tsrc, tdst, edge_index[0], edge_index[1], m16)
    return _final(partials[0], partials[1], tsrc, tdst,
                  m16.reshape(1, 16), bias.reshape(1, HC))


# speedup vs baseline: 52.4570x; 52.4570x over previous
import dataclasses
import functools

import jax
import jax.numpy as jnp
from jax import lax
from jax.experimental import pallas as pl
from jax.experimental.pallas import tpu as pltpu
from jax.experimental.pallas import tpu_sc as plsc

N = 10000
E = 320000
D = 128
H = 8
C = 16
HC = H * C
TW = HC + 16

NB = 400
NBLK = N // NB

ECH = 128
NCHUNK = E // ECH
NWORK = 32
_BASE_CH = NCHUNK // NWORK
_REM_CH = NCHUNK - _BASE_CH * NWORK
NPAD = 10240
RPS = NPAD // 16


def _prep_body(x_ref, w_ref, as_ref, ad_ref, tsrc_ref, tdst_ref, pmax_ref):
    h = jnp.dot(x_ref[...], w_ref[...], precision=lax.Precision.HIGHEST)
    h3 = h.reshape(NB, H, C)
    a_s = jnp.sum(h3 * as_ref[...][None], axis=-1)
    a_d = jnp.sum(h3 * ad_ref[...][None], axis=-1)
    hc = h3.transpose(0, 2, 1).reshape(NB, HC)
    zeros8 = jnp.zeros((NB, 8), jnp.float32)
    tsrc_ref[...] = jnp.concatenate([hc, a_s, zeros8], axis=1)
    tdst_ref[...] = jnp.concatenate([a_d, zeros8], axis=1)
    pmax_ref[...] = jnp.concatenate(
        [jnp.max(a_s, axis=0), jnp.max(a_d, axis=0)]).reshape(1, 1, 16)


def _prep(x, W, att_s, att_d):
    return pl.pallas_call(
        _prep_body,
        grid=(NBLK,),
        in_specs=[
            pl.BlockSpec((NB, D), lambda i: (i, 0)),
            pl.BlockSpec((D, HC), lambda i: (0, 0)),
            pl.BlockSpec((H, C), lambda i: (0, 0)),
            pl.BlockSpec((H, C), lambda i: (0, 0)),
        ],
        out_specs=[
            pl.BlockSpec((NB, TW), lambda i: (i, 0)),
            pl.BlockSpec((NB, 16), lambda i: (i, 0)),
            pl.BlockSpec((1, 1, 16), lambda i: (i, 0, 0)),
        ],
        out_shape=[
            jax.ShapeDtypeStruct((N, TW), jnp.float32),
            jax.ShapeDtypeStruct((N, 16), jnp.float32),
            jax.ShapeDtypeStruct((NBLK, 1, 16), jnp.float32),
        ],
    )(x, W, att_s, att_d)


def _sc_body(tsrc_hbm, tdst_hbm, src_hbm, dst_hbm, m_hbm, out_hbm,
             idx_s, idx_d, rows, drows, mvec, acc):
    cid = lax.axis_index("c")
    sid = lax.axis_index("s")
    wid = sid * 2 + cid

    zero16 = jnp.zeros((16,), jnp.float32)

    @pl.loop(0, ECH)
    def _(r):
        for k in range(TW // 16):
            rows[r, pl.ds(16 * k, 16)] = zero16

    @pl.loop(0, 5)
    def _(z):
        pltpu.sync_copy(rows.at[pl.ds(0, RPS // 5)],
                        acc.at[pl.ds(sid * RPS + z * (RPS // 5), RPS // 5)])

    plsc.subcore_barrier()

    pltpu.sync_copy(m_hbm, mvec)
    m = mvec[...]
    pat = lax.rem(lax.iota(jnp.int32, 16), jnp.full((16,), 8, jnp.int32))
    colv = pat + jnp.full((16,), HC, jnp.int32)

    nch = _BASE_CH + jnp.where(wid < _REM_CH, 1, 0)

    @pl.loop(0, nch)
    def _(j):
        off = (wid + NWORK * j) * ECH
        pltpu.sync_copy(src_hbm.at[pl.ds(off, ECH)], idx_s)
        pltpu.sync_copy(dst_hbm.at[pl.ds(off, ECH)], idx_d)
        pltpu.sync_copy(tsrc_hbm.at[idx_s], rows)
        pltpu.sync_copy(tdst_hbm.at[idx_d], drows)

        @pl.loop(0, ECH)
        def _(e):
            a_s = rows[e, pl.ds(HC, 16)]
            a_d = drows[e, pl.ds(0, 16)]
            t = a_s + a_d
            lrelu = jnp.maximum(t, 0.2 * t)
            wv = jnp.exp(lrelu - m)
            rows[e, pl.ds(HC, 16)] = wv
            rowv = jnp.full((16,), e, jnp.int32)
            wt = plsc.load_gather(rows, [rowv, colv])
            for k in range(H):
                sl = pl.ds(16 * k, 16)
                rows[e, sl] = rows[e, sl] * wt

        pltpu.sync_copy(rows, acc.at[idx_d], add=True)

    plsc.subcore_barrier()

    @pl.loop(0, 5)
    def _(z):
        r0 = sid * RPS + z * (RPS // 5)
        pltpu.sync_copy(acc.at[pl.ds(r0, RPS // 5)],
                        out_hbm.at[cid, pl.ds(r0, RPS // 5)])


_SC_CP = pltpu.CompilerParams(needs_layout_passes=False,
                              use_tc_tiling_on_sc=False)


def _sc_edges(tsrc, tdst, src, dst, m16):
    return pl.kernel(
        _sc_body,
        compiler_params=_SC_CP,
        out_type=jax.ShapeDtypeStruct((2, NPAD, TW), jnp.float32),
        mesh=plsc.VectorSubcoreMesh(core_axis_name="c", subcore_axis_name="s"),
        scratch_types=[
            pltpu.VMEM((ECH,), jnp.int32),
            pltpu.VMEM((ECH,), jnp.int32),
            pltpu.VMEM((ECH, TW), jnp.float32),
            pltpu.VMEM((ECH, 16), jnp.float32),
            pltpu.VMEM((16,), jnp.float32),
            pltpu.VMEM_SHARED((NPAD, TW), jnp.float32),
        ],
    )(tsrc, tdst, src, dst, m16)


def _final_body(p0_ref, p1_ref, tsrc_ref, tdst_ref, m_ref, b_ref, o_ref):
    tsrc = tsrc_ref[...]
    asrc = tsrc[:, HC:HC + 8]
    adst = tdst_ref[...][:, :8]
    t = asrc + adst
    lrelu = jnp.maximum(t, 0.2 * t)
    wself = jnp.exp(lrelu - m_ref[0, :8][None, :])
    p0 = p0_ref[...]
    p1 = p1_ref[...]
    num = p0[:, :HC] + p1[:, :HC] + tsrc[:, :HC] * jnp.tile(wself, (1, C))
    den = p0[:, HC:HC + 8] + p1[:, HC:HC + 8] + wself + 1e-16
    outc = num / jnp.tile(den, (1, C))
    o_ref[...] = (outc.reshape(NB, C, H).transpose(0, 2, 1).reshape(NB, HC)
                  + b_ref[0][None, :])


def _final(p0, p1, tsrc, tdst, m16, bias):
    return pl.pallas_call(
        _final_body,
        grid=(NBLK,),
        in_specs=[
            pl.BlockSpec((NB, TW), lambda i: (i, 0)),
            pl.BlockSpec((NB, TW), lambda i: (i, 0)),
            pl.BlockSpec((NB, TW), lambda i: (i, 0)),
            pl.BlockSpec((NB, 16), lambda i: (i, 0)),
            pl.BlockSpec((1, 16), lambda i: (0, 0)),
            pl.BlockSpec((1, HC), lambda i: (0, 0)),
        ],
        out_specs=pl.BlockSpec((NB, HC), lambda i: (i, 0)),
        out_shape=jax.ShapeDtypeStruct((N, HC), jnp.float32),
    )(p0, p1, tsrc, tdst, m16, bias)


def kernel(x, edge_index, W, att_src, att_dst, bias):
    tsrc, tdst, pmax = _prep(x, W, att_src.reshape(H, C), att_dst.reshape(H, C))
    pm = pmax.reshape(NBLK, 16)
    m8 = jnp.max(pm[:, :8], axis=0) + jnp.max(pm[:, 8:], axis=0)
    m16 = jnp.concatenate([m8, jnp.full((8,), 1e30, jnp.float32)])
    partials = _sc_edges(tsrc, tdst, edge_index[0], edge_index[1], m16)
    return _final(partials[0, :N], partials[1, :N], tsrc, tdst,
                  m16.reshape(1, 16), bias.reshape(1, HC))


# speedup vs baseline: 75.7356x; 1.4438x over previous
import jax
import jax.numpy as jnp
from jax import lax
from jax.experimental import pallas as pl
from jax.experimental.pallas import tpu as pltpu
from jax.experimental.pallas import tpu_sc as plsc

N = 10000
E = 320000
D = 128
H = 8
C = 16
HC = H * C
TW = HC + 16

NB = 400
NBLK = N // NB

ECH = 80
NCHUNK = E // ECH
NWORK = 32
_BASE_CH = NCHUNK // NWORK
NPAD = 10240
RPS = NPAD // 16

_HI = lax.Precision.HIGHEST


def _prep_body(x_ref, wp_ref, as_ref, ad_ref, tsrc_ref, tdst_ref, pmax_ref):
    hc = jnp.dot(x_ref[...], wp_ref[...], precision=_HI)
    a_s = jnp.dot(hc, as_ref[...], precision=_HI)
    a_d = jnp.dot(hc, ad_ref[...], precision=_HI)
    zeros8 = jnp.zeros((NB, 8), jnp.float32)
    tsrc_ref[...] = jnp.concatenate([hc, a_s, zeros8], axis=1)
    tdst_ref[...] = jnp.concatenate([a_d, zeros8], axis=1)
    pmax_ref[...] = jnp.concatenate(
        [jnp.max(a_s, axis=0), jnp.max(a_d, axis=0)]).reshape(1, 1, 16)


def _prep(x, Wp, A_s, A_d):
    return pl.pallas_call(
        _prep_body,
        grid=(NBLK,),
        in_specs=[
            pl.BlockSpec((NB, D), lambda i: (i, 0)),
            pl.BlockSpec((D, HC), lambda i: (0, 0)),
            pl.BlockSpec((HC, 8), lambda i: (0, 0)),
            pl.BlockSpec((HC, 8), lambda i: (0, 0)),
        ],
        out_specs=[
            pl.BlockSpec((NB, TW), lambda i: (i, 0)),
            pl.BlockSpec((NB, 16), lambda i: (i, 0)),
            pl.BlockSpec((1, 1, 16), lambda i: (i, 0, 0)),
        ],
        out_shape=[
            jax.ShapeDtypeStruct((N, TW), jnp.float32),
            jax.ShapeDtypeStruct((N, 16), jnp.float32),
            jax.ShapeDtypeStruct((NBLK, 1, 16), jnp.float32),
        ],
    )(x, Wp, A_s, A_d)


def _sc_body(tsrc_hbm, tdst_hbm, src_hbm, dst_hbm, m_hbm, out_hbm,
             is0, id0, rows0, drows0, is1, id1, rows1, drows1,
             mvec, acc, gsem0, gsem1):
    cid = lax.axis_index("c")
    sid = lax.axis_index("s")
    wid = sid * 2 + cid

    bufs = ((is0, id0, rows0, drows0, gsem0),
            (is1, id1, rows1, drows1, gsem1))

    def fire(c, b):
        is_, id_, rows_, drows_, sem = bufs[b]
        off = c * ECH
        pltpu.sync_copy(src_hbm.at[pl.ds(off, ECH)], is_)
        pltpu.sync_copy(dst_hbm.at[pl.ds(off, ECH)], id_)
        pltpu.async_copy(tsrc_hbm.at[is_], rows_, sem)
        pltpu.async_copy(tdst_hbm.at[id_], drows_, sem)

    def drain(b):
        is_, id_, rows_, drows_, sem = bufs[b]
        pltpu.make_async_copy(tsrc_hbm.at[is_], rows_, sem).wait()
        pltpu.make_async_copy(tdst_hbm.at[id_], drows_, sem).wait()

    zero16 = jnp.zeros((16,), jnp.float32)

    @pl.loop(0, ECH)
    def _(r):
        for k in range(TW // 16):
            rows0[r, pl.ds(16 * k, 16)] = zero16

    @pl.loop(0, RPS // ECH)
    def _(z):
        pltpu.sync_copy(rows0.at[pl.ds(0, ECH)],
                        acc.at[pl.ds(sid * RPS + z * ECH, ECH)])

    plsc.subcore_barrier()

    pltpu.sync_copy(m_hbm, mvec)
    m = mvec[...]
    pat = lax.rem(lax.iota(jnp.int32, 16), jnp.full((16,), 8, jnp.int32))
    colv = pat + jnp.full((16,), HC, jnp.int32)

    def compute(b):
        rows_, drows_ = bufs[b][2], bufs[b][3]

        @pl.loop(0, ECH)
        def _(e):
            a_s = rows_[e, pl.ds(HC, 16)]
            a_d = drows_[e, pl.ds(0, 16)]
            t = a_s + a_d
            lrelu = jnp.maximum(t, 0.2 * t)
            wv = jnp.exp(lrelu - m)
            rows_[e, pl.ds(HC, 16)] = wv
            rowv = jnp.full((16,), e, jnp.int32)
            wt = plsc.load_gather(rows_, [rowv, colv])
            for k in range(H):
                sl = pl.ds(16 * k, 16)
                rows_[e, sl] = rows_[e, sl] * wt

    def scatter(b):
        id_, rows_ = bufs[b][1], bufs[b][2]
        pltpu.sync_copy(rows_, acc.at[id_], add=True)

    fire(wid, 0)
    fire(wid + NWORK, 1)

    @pl.loop(0, _BASE_CH // 2)
    def _(t):
        k0 = 2 * t
        for b in range(2):
            k = k0 + b
            drain(b)
            compute(b)
            scatter(b)

            @pl.when(k + 2 < _BASE_CH)
            def _():
                fire(wid + NWORK * (k + 2), b)

    drain(0)
    compute(0)
    scatter(0)

    plsc.subcore_barrier()

    @pl.loop(0, 5)
    def _(z):
        r0 = sid * RPS + z * (RPS // 5)
        pltpu.sync_copy(acc.at[pl.ds(r0, RPS // 5)],
                        out_hbm.at[cid, pl.ds(r0, RPS // 5)])


_SC_CP = pltpu.CompilerParams(needs_layout_passes=False,
                              use_tc_tiling_on_sc=False)


def _sc_edges(tsrc, tdst, src, dst, m16):
    return pl.kernel(
        _sc_body,
        compiler_params=_SC_CP,
        out_type=jax.ShapeDtypeStruct((2, NPAD, TW), jnp.float32),
        mesh=plsc.VectorSubcoreMesh(core_axis_name="c", subcore_axis_name="s"),
        scratch_types=[
            pltpu.VMEM((ECH,), jnp.int32),
            pltpu.VMEM((ECH,), jnp.int32),
            pltpu.VMEM((ECH, TW), jnp.float32),
            pltpu.VMEM((ECH, 16), jnp.float32),
            pltpu.VMEM((ECH,), jnp.int32),
            pltpu.VMEM((ECH,), jnp.int32),
            pltpu.VMEM((ECH, TW), jnp.float32),
            pltpu.VMEM((ECH, 16), jnp.float32),
            pltpu.VMEM((16,), jnp.float32),
            pltpu.VMEM_SHARED((NPAD, TW), jnp.float32),
            pltpu.SemaphoreType.DMA,
            pltpu.SemaphoreType.DMA,
        ],
    )(tsrc, tdst, src, dst, m16)


def _final_body(p_ref, tsrc_ref, tdst_ref, m_ref, b_ref, perm_ref, o_ref):
    tsrc = tsrc_ref[...]
    asrc = tsrc[:, HC:HC + 8]
    adst = tdst_ref[...][:, :8]
    t = asrc + adst
    lrelu = jnp.maximum(t, 0.2 * t)
    wself = jnp.exp(lrelu - m_ref[0, :8][None, :])
    p0 = p_ref[0]
    p1 = p_ref[1]
    num = p0[:, :HC] + p1[:, :HC] + tsrc[:, :HC] * jnp.tile(wself, (1, C))
    den = p0[:, HC:HC + 8] + p1[:, HC:HC + 8] + wself + 1e-16
    outc = num / jnp.tile(den, (1, C))
    o_ref[...] = (jnp.dot(outc, perm_ref[...], precision=_HI)
                  + b_ref[0][None, :])


def _final(p, tsrc, tdst, m16, bias, P):
    return pl.pallas_call(
        _final_body,
        grid=(NBLK,),
        in_specs=[
            pl.BlockSpec((2, NB, TW), lambda i: (0, i, 0)),
            pl.BlockSpec((NB, TW), lambda i: (i, 0)),
            pl.BlockSpec((NB, 16), lambda i: (i, 0)),
            pl.BlockSpec((1, 16), lambda i: (0, 0)),
            pl.BlockSpec((1, HC), lambda i: (0, 0)),
            pl.BlockSpec((HC, HC), lambda i: (0, 0)),
        ],
        out_specs=pl.BlockSpec((NB, HC), lambda i: (i, 0)),
        out_shape=jax.ShapeDtypeStruct((N, HC), jnp.float32),
    )(p, tsrc, tdst, m16, bias, P)


def kernel(x, edge_index, W, att_src, att_dst, bias):
    idx = jnp.arange(HC, dtype=jnp.int32)
    cmajor_of = (idx % 8) * C + idx // 8
    Wp = W[:, cmajor_of]
    onehot = jax.nn.one_hot(idx % 8, 8, dtype=jnp.float32)
    A_s = att_src.reshape(H, C).T.reshape(HC, 1) * onehot
    A_d = att_dst.reshape(H, C).T.reshape(HC, 1) * onehot
    P = jax.nn.one_hot(cmajor_of, HC, dtype=jnp.float32)

    tsrc, tdst, pmax = _prep(x, Wp, A_s, A_d)
    pm = pmax.reshape(NBLK, 16)
    m8 = jnp.max(pm[:, :8], axis=0) + jnp.max(pm[:, 8:], axis=0)
    m16 = jnp.concatenate([m8, jnp.full((8,), 1e30, jnp.float32)])
    partials = _sc_edges(tsrc, tdst, edge_index[0], edge_index[1], m16)
    return _final(partials, tsrc, tdst,
                  m16.reshape(1, 16), bias.reshape(1, HC), P)


# speedup vs baseline: 108.4145x; 1.4315x over previous
import jax
import jax.numpy as jnp
from jax import lax
from jax.experimental import pallas as pl
from jax.experimental.pallas import tpu as pltpu
from jax.experimental.pallas import tpu_sc as plsc

N = 10000
E = 320000
D = 128
H = 8
C = 16
HC = H * C
TW = HC + 16

NB = 400
NBLK = N // NB

ECH = 80
NCHUNK = E // ECH
NWORK = 32
_BASE_CH = NCHUNK // NWORK
NPAD = 10240
RPS = NPAD // 16

_HI = lax.Precision.HIGHEST


def _prep_body(x_ref, wp_ref, as_ref, ad_ref, tsrc_ref, tdst_ref, pmax_ref):
    hc = jnp.dot(x_ref[...], wp_ref[...], precision=_HI)
    a_s = jnp.dot(hc, as_ref[...], precision=_HI)
    a_d = jnp.dot(hc, ad_ref[...], precision=_HI)
    zeros8 = jnp.zeros((NB, 8), jnp.float32)
    tsrc_ref[...] = jnp.concatenate([hc, a_s, zeros8], axis=1)
    tdst_ref[...] = jnp.concatenate([a_d, zeros8], axis=1)
    pmax_ref[...] = jnp.concatenate(
        [jnp.max(a_s, axis=0), jnp.max(a_d, axis=0)]).reshape(1, 1, 16)


def _prep(x, Wp, A_s, A_d):
    return pl.pallas_call(
        _prep_body,
        grid=(NBLK,),
        in_specs=[
            pl.BlockSpec((NB, D), lambda i: (i, 0)),
            pl.BlockSpec((D, HC), lambda i: (0, 0)),
            pl.BlockSpec((HC, 8), lambda i: (0, 0)),
            pl.BlockSpec((HC, 8), lambda i: (0, 0)),
        ],
        out_specs=[
            pl.BlockSpec((NB, TW), lambda i: (i, 0)),
            pl.BlockSpec((NB, 16), lambda i: (i, 0)),
            pl.BlockSpec((1, 1, 16), lambda i: (i, 0, 0)),
        ],
        out_shape=[
            jax.ShapeDtypeStruct((N, TW), jnp.float32),
            jax.ShapeDtypeStruct((N, 16), jnp.float32),
            jax.ShapeDtypeStruct((NBLK, 1, 16), jnp.float32),
        ],
    )(x, Wp, A_s, A_d)


def _sc_body(tsrc_hbm, tdst_hbm, src_hbm, dst_hbm, m_hbm, out_hbm,
             is0, id0, rows0, drows0, is1, id1, rows1, drows1,
             mvec, acc, gsem0, gsem1):
    cid = lax.axis_index("c")
    sid = lax.axis_index("s")
    wid = sid * 2 + cid

    bufs = ((is0, id0, rows0, drows0, gsem0),
            (is1, id1, rows1, drows1, gsem1))

    def fire(c, b):
        is_, id_, rows_, drows_, sem = bufs[b]
        off = c * ECH
        pltpu.sync_copy(src_hbm.at[pl.ds(off, ECH)], is_)
        pltpu.sync_copy(dst_hbm.at[pl.ds(off, ECH)], id_)
        pltpu.async_copy(tsrc_hbm.at[is_], rows_, sem)
        pltpu.async_copy(tdst_hbm.at[id_], drows_, sem)

    def drain(b):
        is_, id_, rows_, drows_, sem = bufs[b]
        pltpu.make_async_copy(tsrc_hbm.at[is_], rows_, sem).wait()
        pltpu.make_async_copy(tdst_hbm.at[id_], drows_, sem).wait()

    zero16 = jnp.zeros((16,), jnp.float32)

    @pl.loop(0, ECH)
    def _(r):
        for k in range(TW // 16):
            rows0[r, pl.ds(16 * k, 16)] = zero16

    @pl.loop(0, RPS // ECH)
    def _(z):
        pltpu.sync_copy(rows0.at[pl.ds(0, ECH)],
                        acc.at[pl.ds(sid * RPS + z * ECH, ECH)])

    plsc.subcore_barrier()

    pltpu.sync_copy(m_hbm, mvec)
    m = mvec[...]
    pat = lax.rem(lax.iota(jnp.int32, 16), jnp.full((16,), 8, jnp.int32))
    colv = pat + jnp.full((16,), HC, jnp.int32)

    def compute(b):
        rows_, drows_ = bufs[b][2], bufs[b][3]

        @plsc.parallel_loop(0, ECH, unroll=4)
        def _(e):
            a_s = rows_[e, pl.ds(HC, 16)]
            a_d = drows_[e, pl.ds(0, 16)]
            t = a_s + a_d
            lrelu = jnp.maximum(t, 0.2 * t)
            wv = jnp.exp(lrelu - m)
            rows_[e, pl.ds(HC, 16)] = wv
            rowv = jnp.full((16,), e, jnp.int32)
            wt = plsc.load_gather(rows_, [rowv, colv])
            for k in range(H):
                sl = pl.ds(16 * k, 16)
                rows_[e, sl] = rows_[e, sl] * wt

    def scatter(b):
        id_, rows_ = bufs[b][1], bufs[b][2]
        pltpu.sync_copy(rows_, acc.at[id_], add=True)

    fire(wid, 0)
    fire(wid + NWORK, 1)

    @pl.loop(0, _BASE_CH // 2)
    def _(t):
        k0 = 2 * t
        for b in range(2):
            k = k0 + b
            drain(b)
            compute(b)
            scatter(b)

            @pl.when(k + 2 < _BASE_CH)
            def _():
                fire(wid + NWORK * (k + 2), b)

    drain(0)
    compute(0)
    scatter(0)

    plsc.subcore_barrier()

    @pl.loop(0, 5)
    def _(z):
        r0 = sid * RPS + z * (RPS // 5)
        pltpu.sync_copy(acc.at[pl.ds(r0, RPS // 5)],
                        out_hbm.at[cid, pl.ds(r0, RPS // 5)])


_SC_CP = pltpu.CompilerParams(needs_layout_passes=False,
                              use_tc_tiling_on_sc=False)


def _sc_edges(tsrc, tdst, src, dst, m16):
    return pl.kernel(
        _sc_body,
        compiler_params=_SC_CP,
        out_type=jax.ShapeDtypeStruct((2, NPAD, TW), jnp.float32),
        mesh=plsc.VectorSubcoreMesh(core_axis_name="c", subcore_axis_name="s"),
        scratch_types=[
            pltpu.VMEM((ECH,), jnp.int32),
            pltpu.VMEM((ECH,), jnp.int32),
            pltpu.VMEM((ECH, TW), jnp.float32),
            pltpu.VMEM((ECH, 16), jnp.float32),
            pltpu.VMEM((ECH,), jnp.int32),
            pltpu.VMEM((ECH,), jnp.int32),
            pltpu.VMEM((ECH, TW), jnp.float32),
            pltpu.VMEM((ECH, 16), jnp.float32),
            pltpu.VMEM((16,), jnp.float32),
            pltpu.VMEM_SHARED((NPAD, TW), jnp.float32),
            pltpu.SemaphoreType.DMA,
            pltpu.SemaphoreType.DMA,
        ],
    )(tsrc, tdst, src, dst, m16)


def _final_body(p_ref, tsrc_ref, tdst_ref, m_ref, b_ref, perm_ref, o_ref):
    tsrc = tsrc_ref[...]
    asrc = tsrc[:, HC:HC + 8]
    adst = tdst_ref[...][:, :8]
    t = asrc + adst
    lrelu = jnp.maximum(t, 0.2 * t)
    wself = jnp.exp(lrelu - m_ref[0, :8][None, :])
    p0 = p_ref[0]
    p1 = p_ref[1]
    num = p0[:, :HC] + p1[:, :HC] + tsrc[:, :HC] * jnp.tile(wself, (1, C))
    den = p0[:, HC:HC + 8] + p1[:, HC:HC + 8] + wself + 1e-16
    outc = num / jnp.tile(den, (1, C))
    o_ref[...] = (jnp.dot(outc, perm_ref[...], precision=_HI)
                  + b_ref[0][None, :])


def _final(p, tsrc, tdst, m16, bias, P):
    return pl.pallas_call(
        _final_body,
        grid=(NBLK,),
        in_specs=[
            pl.BlockSpec((2, NB, TW), lambda i: (0, i, 0)),
            pl.BlockSpec((NB, TW), lambda i: (i, 0)),
            pl.BlockSpec((NB, 16), lambda i: (i, 0)),
            pl.BlockSpec((1, 16), lambda i: (0, 0)),
            pl.BlockSpec((1, HC), lambda i: (0, 0)),
            pl.BlockSpec((HC, HC), lambda i: (0, 0)),
        ],
        out_specs=pl.BlockSpec((NB, HC), lambda i: (i, 0)),
        out_shape=jax.ShapeDtypeStruct((N, HC), jnp.float32),
    )(p, tsrc, tdst, m16, bias, P)


def kernel(x, edge_index, W, att_src, att_dst, bias):
    idx = jnp.arange(HC, dtype=jnp.int32)
    cmajor_of = (idx % 8) * C + idx // 8
    Wp = W[:, cmajor_of]
    onehot = jax.nn.one_hot(idx % 8, 8, dtype=jnp.float32)
    A_s = att_src.reshape(H, C).T.reshape(HC, 1) * onehot
    A_d = att_dst.reshape(H, C).T.reshape(HC, 1) * onehot
    P = jax.nn.one_hot(cmajor_of, HC, dtype=jnp.float32)

    tsrc, tdst, pmax = _prep(x, Wp, A_s, A_d)
    pm = pmax.reshape(NBLK, 16)
    m8 = jnp.max(pm[:, :8], axis=0) + jnp.max(pm[:, 8:], axis=0)
    m16 = jnp.concatenate([m8, jnp.full((8,), 1e30, jnp.float32)])
    partials = _sc_edges(tsrc, tdst, edge_index[0], edge_index[1], m16)
    return _final(partials, tsrc, tdst,
                  m16.reshape(1, 16), bias.reshape(1, HC), P)
